# baseline (speedup 1.0000x reference)
import functools

import jax
import jax.numpy as jnp
from jax import lax
from jax.experimental import pallas as pl
from jax.experimental.pallas import tpu as pltpu
from jax.experimental.pallas import tpu_sc as plsc

_BS, _TPD, _D = 16384, 4, 32
_G, _K = 4, 512
_GK = _G * _K
_ROWS = 8192
_NBLK = _BS // _ROWS

_NW = 32
_BPW = (_BS * _G) // _NW
_HPW = (_BS * _G) // 16

_KT = 128
_NKT = _K // _KT


def _tc_body(x_ref, emb_ref, idx_ref):
    xb = x_ref[...]
    idx_cols = []
    lane = jax.lax.broadcasted_iota(jnp.int32, (_ROWS, _KT), 1)
    for g in range(_G):
        xg = xb[:, g * _D:(g + 1) * _D]
        x_sq = jnp.sum(xg * xg, axis=1, keepdims=True)
        m = None
        for kt in range(_NKT):
            egt = emb_ref[g, kt * _KT:(kt + 1) * _KT, :]
            cross = jax.lax.dot_general(
                xg, egt, (((1,), (1,)), ((), ())),
                preferred_element_type=jnp.float32)
            e_sq = jnp.sum(egt * egt, axis=1)
            d2t = jnp.maximum(x_sq - 2.0 * cross + e_sq[None, :], 0.0)
            if m is None:
                m, gi = d2t, lane
            else:
                take = d2t < m
                m = jnp.where(take, d2t, m)
                gi = jnp.where(take, lane + kt * _KT, gi)
        gmin = jnp.min(m, axis=1, keepdims=True)
        cand = jnp.where(m == gmin, gi, _GK)
        idxg = jnp.min(cand, axis=1).astype(jnp.int32)
        idx_cols.append(idxg.reshape(_ROWS, 1))
    idx_ref[...] = jnp.concatenate(idx_cols, axis=1)


def _tc_indices(x2, embeddings):
    return pl.pallas_call(
        _tc_body,
        grid=(_NBLK,),
        in_specs=[
            pl.BlockSpec((_ROWS, _G * _D), lambda i: (i, 0)),
            pl.BlockSpec((_G, _K, _D), lambda i: (0, 0, 0)),
        ],
        out_specs=pl.BlockSpec((_ROWS, _G), lambda i: (i, 0)),
        out_shape=jax.ShapeDtypeStruct((_BS, _G), jnp.int32),
    )(x2, embeddings)


_SC_MESH = plsc.VectorSubcoreMesh(core_axis_name="c", subcore_axis_name="s")


@functools.partial(
    pl.kernel,
    mesh=_SC_MESH,
    compiler_params=pltpu.CompilerParams(
        needs_layout_passes=False, use_tc_tiling_on_sc=False),
    out_type=[
        jax.ShapeDtypeStruct((_BS, _G * _D), jnp.float32),
        jax.ShapeDtypeStruct((_GK,), jnp.float32),
    ],
    scratch_types=[
        pltpu.VMEM((_BPW,), jnp.int32),
        pltpu.VMEM((_BPW // 2, _D), jnp.float32),
        pltpu.VMEM((_BPW // 8, 128), jnp.float32),
        pltpu.VMEM((_HPW,), jnp.int32),
        pltpu.VMEM((16 * _GK,), jnp.float32),
        pltpu.VMEM((_GK,), jnp.float32),
        pltpu.VMEM((128,), jnp.float32),
        pltpu.VMEM((128,), jnp.float32),
        pltpu.VMEM_SHARED((16, _GK), jnp.float32),
        pltpu.SemaphoreType.DMA,
    ],
)
def _sc_gather_count(idx_hbm, table_hbm, cnt_hbm, xq_hbm, cnt_out_hbm,
                     idx_v, rows_v, rows128_v, hidx_v, hist_v, hsum_v,
                     t1_v, t2_v, shared_hist, sem):
    c = lax.axis_index("c")
    s = lax.axis_index("s")
    wid = c * 16 + s
    base = wid * _BPW

    lane = lax.broadcasted_iota(jnp.int32, (16,), 0)
    offs = (lane % 4) * _K
    ones = jnp.full((16,), 1.0, dtype=jnp.float32)
    zeros = jnp.zeros((16,), dtype=jnp.float32)

    pltpu.sync_copy(idx_hbm.at[pl.ds(base, _BPW)], idx_v)

    def _add_offs(k, _):
        idx_v[pl.ds(k * 16, 16)] += offs
        return 0
    lax.fori_loop(0, _BPW // 16, _add_offs, 0)

    copies0 = []
    for r in range(8):
        copies0.append(pltpu.async_copy(
            table_hbm.at[idx_v.at[pl.ds(r * 128, 128)]],
            rows_v.at[pl.ds(r * 128, 128)], sem))

    @pl.when(c == 0)
    def _():
        def _zero(k, _):
            hist_v[pl.ds(k * 16, 16)] = zeros
            return 0
        lax.fori_loop(0, 16 * _GK // 16, _zero, 0)

        pltpu.sync_copy(idx_hbm.at[pl.ds(s * _HPW, _HPW)], hidx_v)

        def _hist(k, _):
            binv = hidx_v[pl.ds(k * 16, 16)] + offs
            plsc.addupdate_scatter(hist_v, [lane * _GK + binv], ones)
            return 0
        lax.fori_loop(0, _HPW // 16, _hist, 0)

        def _red(k, _):
            acc = zeros
            for l in range(16):
                acc += hist_v[pl.ds(l * _GK + k * 16, 16)]
            hsum_v[pl.ds(k * 16, 16)] = acc
            return 0
        lax.fori_loop(0, _GK // 16, _red, 0)

        pltpu.sync_copy(hsum_v, shared_hist.at[s])

    for h in range(2):
        if h == 1:
            copies0 = []
            for r in range(8):
                copies0.append(pltpu.async_copy(
                    table_hbm.at[idx_v.at[pl.ds((8 + r) * 128, 128)]],
                    rows_v.at[pl.ds(r * 128, 128)], sem))
        for cp in copies0:
            cp.wait()

        def _repack(r, _):
            for col in range(8):
                rows128_v[r, pl.ds(col * 16, 16)] = (
                    rows_v[4 * r + col // 2, pl.ds((col % 2) * 16, 16)])
            return 0
        lax.fori_loop(0, _BPW // 8, _repack, 0)
        pltpu.sync_copy(
            rows128_v, xq_hbm.at[pl.ds(wid * (_BPW // 4) + h * (_BPW // 8),
                                       _BPW // 8)])

    @pl.when(c == 0)
    def _():
        plsc.subcore_barrier()
        pltpu.sync_copy(cnt_hbm.at[pl.ds(s * 128, 128)], t2_v)
        for r in range(16):
            pltpu.sync_copy(shared_hist.at[r, pl.ds(s * 128, 128)], t1_v)

            def _addc(k, _):
                t2_v[pl.ds(k * 16, 16)] += t1_v[pl.ds(k * 16, 16)]
                return 0
            lax.fori_loop(0, 8, _addc, 0)
        pltpu.sync_copy(t2_v, cnt_out_hbm.at[pl.ds(s * 128, 128)])


@jax.jit
def kernel(x, embeddings, count):
    x2 = x.reshape(_BS, _G * _D)
    idx = _tc_indices(x2, embeddings)
    xq2, cnt = _sc_gather_count(
        idx.reshape(_BS * _G), embeddings.reshape(_GK, _D),
        count.reshape(_GK))
    return xq2.reshape(_BS, _TPD, _D), idx, cnt.reshape(_G, _K)


# speedup vs baseline: 1.2119x; 1.2119x over previous
import functools

import jax
import jax.numpy as jnp
from jax import lax
from jax.experimental import pallas as pl
from jax.experimental.pallas import tpu as pltpu
from jax.experimental.pallas import tpu_sc as plsc

_BS, _TPD, _D = 16384, 4, 32
_G, _K = 4, 512
_GK = _G * _K
_ROWS = 4096
_NBLK = _BS // _ROWS

_NW = 32
_BPW = (_BS * _G) // _NW
_HPW = (_BS * _G) // 16

_KT = 128
_NKT = _K // _KT


def _tc_body(x_ref, emb_ref, idx_ref):
    xb = x_ref[...]
    idx_cols = []
    lane = jax.lax.broadcasted_iota(jnp.int32, (_ROWS, _KT), 1)
    for g in range(_G):
        xg = xb[:, g * _D:(g + 1) * _D]
        x_sq = jnp.sum(xg * xg, axis=1, keepdims=True)
        m = None
        for kt in range(_NKT):
            egt = emb_ref[g, kt * _KT:(kt + 1) * _KT, :]
            cross = jax.lax.dot_general(
                xg, egt, (((1,), (1,)), ((), ())),
                preferred_element_type=jnp.float32)
            e_sq = jnp.sum(egt * egt, axis=1)
            d2t = jnp.maximum(x_sq - 2.0 * cross + e_sq[None, :], 0.0)
            if m is None:
                m, gi = d2t, lane
            else:
                take = d2t < m
                m = jnp.where(take, d2t, m)
                gi = jnp.where(take, lane + kt * _KT, gi)
        gmin = jnp.min(m, axis=1, keepdims=True)
        cand = jnp.where(m == gmin, gi, _GK)
        idxg = jnp.min(cand, axis=1).astype(jnp.int32)
        idx_cols.append(idxg.reshape(_ROWS, 1))
    idx_ref[...] = jnp.concatenate(idx_cols, axis=1)


def _tc_indices(x2, embeddings):
    return pl.pallas_call(
        _tc_body,
        grid=(_NBLK,),
        in_specs=[
            pl.BlockSpec((_ROWS, _G * _D), lambda i: (i, 0)),
            pl.BlockSpec((_G, _K, _D), lambda i: (0, 0, 0)),
        ],
        out_specs=pl.BlockSpec((_ROWS, _G), lambda i: (i, 0)),
        out_shape=jax.ShapeDtypeStruct((_BS, _G), jnp.int32),
    )(x2, embeddings)


_SC_MESH = plsc.VectorSubcoreMesh(core_axis_name="c", subcore_axis_name="s")


@functools.partial(
    pl.kernel,
    mesh=_SC_MESH,
    compiler_params=pltpu.CompilerParams(
        needs_layout_passes=False, use_tc_tiling_on_sc=False),
    out_type=[
        jax.ShapeDtypeStruct((_BS, _G * _D), jnp.float32),
        jax.ShapeDtypeStruct((_GK,), jnp.float32),
    ],
    scratch_types=[
        pltpu.VMEM((_BPW,), jnp.int32),
        pltpu.VMEM((_BPW // 2, _D), jnp.float32),
        pltpu.VMEM((_BPW // 8, 128), jnp.float32),
        pltpu.VMEM((_HPW,), jnp.int32),
        pltpu.VMEM((16 * _GK,), jnp.float32),
        pltpu.VMEM((_GK,), jnp.float32),
        pltpu.VMEM((128,), jnp.float32),
        pltpu.VMEM((128,), jnp.float32),
        pltpu.VMEM_SHARED((16, _GK), jnp.float32),
        pltpu.SemaphoreType.DMA,
    ],
)
def _sc_gather_count(idx_hbm, table_hbm, cnt_hbm, xq_hbm, cnt_out_hbm,
                     idx_v, rows_v, rows128_v, hidx_v, hist_v, hsum_v,
                     t1_v, t2_v, shared_hist, sem):
    c = lax.axis_index("c")
    s = lax.axis_index("s")
    wid = c * 16 + s
    base = wid * _BPW

    lane = lax.broadcasted_iota(jnp.int32, (16,), 0)
    offs = (lane % 4) * _K
    ones = jnp.full((16,), 1.0, dtype=jnp.float32)
    zeros = jnp.zeros((16,), dtype=jnp.float32)

    pltpu.sync_copy(idx_hbm.at[pl.ds(base, _BPW)], idx_v)

    def _add_offs(k, _):
        idx_v[pl.ds(k * 16, 16)] += offs
        return 0
    lax.fori_loop(0, _BPW // 16, _add_offs, 0)

    copies0 = []
    for r in range(8):
        copies0.append(pltpu.async_copy(
            table_hbm.at[idx_v.at[pl.ds(r * 128, 128)]],
            rows_v.at[pl.ds(r * 128, 128)], sem))

    @pl.when(c == 0)
    def _():
        def _zero(k, _):
            hist_v[pl.ds(k * 16, 16)] = zeros
            return 0
        lax.fori_loop(0, 16 * _GK // 16, _zero, 0)

        pltpu.sync_copy(idx_hbm.at[pl.ds(s * _HPW, _HPW)], hidx_v)

        def _hist(k, _):
            binv = hidx_v[pl.ds(k * 16, 16)] + offs
            plsc.addupdate_scatter(hist_v, [lane * _GK + binv], ones)
            return 0
        lax.fori_loop(0, _HPW // 16, _hist, 0)

        def _red(k, _):
            acc = zeros
            for l in range(16):
                acc += hist_v[pl.ds(l * _GK + k * 16, 16)]
            hsum_v[pl.ds(k * 16, 16)] = acc
            return 0
        lax.fori_loop(0, _GK // 16, _red, 0)

        pltpu.sync_copy(hsum_v, shared_hist.at[s])

    for h in range(2):
        if h == 1:
            copies0 = []
            for r in range(8):
                copies0.append(pltpu.async_copy(
                    table_hbm.at[idx_v.at[pl.ds((8 + r) * 128, 128)]],
                    rows_v.at[pl.ds(r * 128, 128)], sem))
        for cp in copies0:
            cp.wait()

        def _repack(r, _):
            for col in range(8):
                rows128_v[r, pl.ds(col * 16, 16)] = (
                    rows_v[4 * r + col // 2, pl.ds((col % 2) * 16, 16)])
            return 0
        lax.fori_loop(0, _BPW // 8, _repack, 0)
        pltpu.sync_copy(
            rows128_v, xq_hbm.at[pl.ds(wid * (_BPW // 4) + h * (_BPW // 8),
                                       _BPW // 8)])

    @pl.when(c == 0)
    def _():
        plsc.subcore_barrier()
        pltpu.sync_copy(cnt_hbm.at[pl.ds(s * 128, 128)], t2_v)
        for r in range(16):
            pltpu.sync_copy(shared_hist.at[r, pl.ds(s * 128, 128)], t1_v)

            def _addc(k, _):
                t2_v[pl.ds(k * 16, 16)] += t1_v[pl.ds(k * 16, 16)]
                return 0
            lax.fori_loop(0, 8, _addc, 0)
        pltpu.sync_copy(t2_v, cnt_out_hbm.at[pl.ds(s * 128, 128)])


@jax.jit
def kernel(x, embeddings, count):
    x2 = x.reshape(_BS, _G * _D)
    idx = _tc_indices(x2, embeddings)
    xq2, cnt = _sc_gather_count(
        idx.reshape(_BS * _G), embeddings.reshape(_GK, _D),
        count.reshape(_GK))
    return xq2.reshape(_BS, _TPD, _D), idx, cnt.reshape(_G, _K)
